# initial kernel scaffold (unmeasured)
import jax
import jax.numpy as jnp
from jax import lax
from jax.experimental import pallas as pl
from jax.experimental.pallas import tpu as pltpu


def kernel(
    x,
):
    def body(*refs):
        pass

    out_shape = jax.ShapeDtypeStruct(..., jnp.float32)
    return pl.pallas_call(body, out_shape=out_shape)(...)


# baseline (device time: 18574 ns/iter reference)
import jax
import jax.numpy as jnp
from jax import lax
from jax.experimental import pallas as pl
from jax.experimental.pallas import tpu as pltpu

N_DEV = 4


def kernel(x):
    m_per, n = x.shape

    def body(x_ref, out_ref, comm_ref, send_sems, recv_sems):
        my_pos = lax.axis_index("i")
        left = (my_pos - 1) % N_DEV
        right = (my_pos + 1) % N_DEV

        barrier_sem = pltpu.get_barrier_semaphore()
        for nbr in [left, right]:
            pl.semaphore_signal(
                barrier_sem, inc=1,
                device_id=(nbr,), device_id_type=pl.DeviceIdType.MESH,
            )
        pl.semaphore_wait(barrier_sem, 2)

        out_ref[pl.ds(my_pos * m_per, m_per), :] = x_ref[:, :]
        comm_ref[0, :, :] = x_ref[:, :]

        for h in range(N_DEV - 1):
            send_slot = h % 2
            recv_slot = (h + 1) % 2
            rdma = pltpu.make_async_remote_copy(
                src_ref=comm_ref.at[send_slot],
                dst_ref=comm_ref.at[recv_slot],
                send_sem=send_sems.at[send_slot],
                recv_sem=recv_sems.at[recv_slot],
                device_id=(right,),
                device_id_type=pl.DeviceIdType.MESH,
            )
            rdma.start()
            rdma.wait()

            origin = (my_pos - h - 1) % N_DEV
            out_ref[pl.ds(origin * m_per, m_per), :] = comm_ref[recv_slot, :, :]

    return pl.pallas_call(
        body,
        out_shape=jax.ShapeDtypeStruct((N_DEV * m_per, n), x.dtype),
        in_specs=[pl.BlockSpec(memory_space=pltpu.VMEM)],
        out_specs=pl.BlockSpec(memory_space=pltpu.VMEM),
        scratch_shapes=[
            pltpu.VMEM((2, m_per, n), x.dtype),
            pltpu.SemaphoreType.DMA((2,)),
            pltpu.SemaphoreType.DMA((2,)),
        ],
        compiler_params=pltpu.CompilerParams(collective_id=0),
    )(x)


# device time: 12244 ns/iter; 1.5170x vs baseline; 1.5170x over previous
import jax
import jax.numpy as jnp
from jax import lax
from jax.experimental import pallas as pl
from jax.experimental.pallas import tpu as pltpu

N_DEV = 4


def kernel(x):
    m, n = x.shape
    h = m // 2

    def body(x_ref, out_ref, send_sems, recv_sems):
        my = lax.axis_index("i")
        left = (my - 1) % N_DEV
        right = (my + 1) % N_DEV
        opp = (my + 2) % N_DEV

        barrier_sem = pltpu.get_barrier_semaphore()
        for nbr in [left, right]:
            pl.semaphore_signal(
                barrier_sem, inc=1,
                device_id=(nbr,), device_id_type=pl.DeviceIdType.MESH,
            )
        pl.semaphore_wait(barrier_sem, 2)

        def copy(src, dst, sem, dev):
            return pltpu.make_async_remote_copy(
                src_ref=src, dst_ref=dst,
                send_sem=send_sems.at[sem], recv_sem=recv_sems.at[sem],
                device_id=(dev,), device_id_type=pl.DeviceIdType.MESH,
            )

        send_r = copy(x_ref, out_ref.at[pl.ds(my * m, m)], 0, right)
        send_l = copy(x_ref, out_ref.at[pl.ds(my * m, m)], 1, left)
        send_r.start()
        send_l.start()

        out_ref[pl.ds(my * m, m), :] = x_ref[:, :]

        recv_l = copy(out_ref.at[pl.ds(left * m, m)],
                      out_ref.at[pl.ds(left * m, m)], 0, right)
        recv_r = copy(out_ref.at[pl.ds(right * m, m)],
                      out_ref.at[pl.ds(right * m, m)], 1, left)

        recv_l.wait_recv()
        fwd_r = copy(out_ref.at[pl.ds(left * m, h)],
                     out_ref.at[pl.ds(left * m, h)], 2, right)
        fwd_r.start()

        recv_r.wait_recv()
        fwd_l = copy(out_ref.at[pl.ds(right * m + h, h)],
                     out_ref.at[pl.ds(right * m + h, h)], 3, left)
        fwd_l.start()

        recv_opp_t = copy(out_ref.at[pl.ds(opp * m, h)],
                          out_ref.at[pl.ds(opp * m, h)], 2, right)
        recv_opp_b = copy(out_ref.at[pl.ds(opp * m + h, h)],
                          out_ref.at[pl.ds(opp * m + h, h)], 3, left)
        recv_opp_t.wait_recv()
        recv_opp_b.wait_recv()

        send_r.wait_send()
        send_l.wait_send()
        fwd_r.wait_send()
        fwd_l.wait_send()

    return pl.pallas_call(
        body,
        out_shape=jax.ShapeDtypeStruct((N_DEV * m, n), x.dtype),
        in_specs=[pl.BlockSpec(memory_space=pltpu.VMEM)],
        out_specs=pl.BlockSpec(memory_space=pltpu.VMEM),
        scratch_shapes=[
            pltpu.SemaphoreType.DMA((4,)),
            pltpu.SemaphoreType.DMA((4,)),
        ],
        compiler_params=pltpu.CompilerParams(collective_id=0),
    )(x)


# device time: 11050 ns/iter; 1.6809x vs baseline; 1.1081x over previous
import jax
import jax.numpy as jnp
from jax import lax
from jax.experimental import pallas as pl
from jax.experimental.pallas import tpu as pltpu

N_DEV = 4


def kernel(x):
    m, n = x.shape
    h = m // 2

    def body(x_ref, out_ref, send_sems, recv_sems):
        my = lax.axis_index("i")
        left = (my - 1) % N_DEV
        right = (my + 1) % N_DEV
        opp = (my + 2) % N_DEV

        barrier_sem = pltpu.get_barrier_semaphore()
        for nbr in [left, right]:
            pl.semaphore_signal(
                barrier_sem, inc=1,
                device_id=(nbr,), device_id_type=pl.DeviceIdType.MESH,
            )
        pl.semaphore_wait(barrier_sem, 2)

        def copy(src, dst, sem, dev):
            return pltpu.make_async_remote_copy(
                src_ref=src, dst_ref=dst,
                send_sem=send_sems.at[sem], recv_sem=recv_sems.at[sem],
                device_id=(dev,), device_id_type=pl.DeviceIdType.MESH,
            )

        my_top = out_ref.at[pl.ds(my * m, h)]
        my_bot = out_ref.at[pl.ds(my * m + h, h)]

        s_top_r = copy(x_ref.at[pl.ds(0, h)], my_top, 0, right)
        s_bot_l = copy(x_ref.at[pl.ds(h, h)], my_bot, 1, left)
        s_bot_r = copy(x_ref.at[pl.ds(h, h)], my_bot, 2, right)
        s_top_l = copy(x_ref.at[pl.ds(0, h)], my_top, 3, left)
        s_top_r.start()
        s_bot_l.start()
        s_bot_r.start()
        s_top_l.start()

        out_ref[pl.ds(my * m, m), :] = x_ref[:, :]

        left_top = out_ref.at[pl.ds(left * m, h)]
        left_bot = out_ref.at[pl.ds(left * m + h, h)]
        right_top = out_ref.at[pl.ds(right * m, h)]
        right_bot = out_ref.at[pl.ds(right * m + h, h)]
        opp_top = out_ref.at[pl.ds(opp * m, h)]
        opp_bot = out_ref.at[pl.ds(opp * m + h, h)]

        copy(left_top, left_top, 0, right).wait_recv()
        f_r = copy(left_top, left_top, 4, right)
        f_r.start()

        copy(right_bot, right_bot, 1, left).wait_recv()
        f_l = copy(right_bot, right_bot, 5, left)
        f_l.start()

        copy(left_bot, left_bot, 2, right).wait_recv()
        copy(right_top, right_top, 3, left).wait_recv()
        copy(opp_top, opp_top, 4, right).wait_recv()
        copy(opp_bot, opp_bot, 5, left).wait_recv()

        s_top_r.wait_send()
        s_bot_l.wait_send()
        s_bot_r.wait_send()
        s_top_l.wait_send()
        f_r.wait_send()
        f_l.wait_send()

    return pl.pallas_call(
        body,
        out_shape=jax.ShapeDtypeStruct((N_DEV * m, n), x.dtype),
        in_specs=[pl.BlockSpec(memory_space=pltpu.VMEM)],
        out_specs=pl.BlockSpec(memory_space=pltpu.VMEM),
        scratch_shapes=[
            pltpu.SemaphoreType.DMA((6,)),
            pltpu.SemaphoreType.DMA((6,)),
        ],
        compiler_params=pltpu.CompilerParams(collective_id=0),
    )(x)
